# baseline (speedup 1.0000x reference)
import functools

import jax
import jax.numpy as jnp
from jax.experimental import pallas as pl
from jax.experimental.pallas import tpu as pltpu

_BN_EPS = 1e-5
_PAD = 8


def _fused_body(x30_ref, w10_ref, b10_ref, x_ref, w11_ref, gamma_ref, beta_ref,
                out_ref,
                w11g_ref, xb_ref, w2_ref, sum_ref, sumsq_ref,
                *, tm, m_total, n_tiles):
    p = pl.program_id(0)
    j = pl.program_id(1)

    @pl.when(jnp.logical_and(p == 0, j == 0))
    def _init():
        g = jax.lax.dot_general(x30_ref[...], w10_ref[...],
                                (((1,), (1,)), ((), ())),
                                preferred_element_type=jnp.float32)
        gate = jax.nn.sigmoid(g + b10_ref[...])
        w11g_ref[...] = (w11_ref[...] * gate).astype(jnp.bfloat16)
        sum_ref[...] = jnp.zeros_like(sum_ref)
        sumsq_ref[...] = jnp.zeros_like(sumsq_ref)
        out_ref[...] = jnp.zeros_like(out_ref)

    def _phase0_work(xb):
        xb_ref[j, :xb.shape[0], :] = xb
        row = jax.lax.broadcasted_iota(jnp.int32, (_PAD, tm), 0)
        xb_ref[j, xb.shape[0]:, :] = jnp.where(row == 0, 1.0, 0.0).astype(jnp.bfloat16)
        y = jnp.dot(w11g_ref[...], xb, preferred_element_type=jnp.float32)
        sum_ref[...] += jnp.sum(y, axis=1, keepdims=True)
        sumsq_ref[...] += jnp.sum(y * y, axis=1, keepdims=True)

    @pl.when(jnp.logical_and(p == 0, j < n_tiles - 1))
    def _phase0_full():
        _phase0_work(x_ref[...].astype(jnp.bfloat16))

    @pl.when(jnp.logical_and(p == 0, j == n_tiles - 1))
    def _phase0_last():
        cols = j * tm + jax.lax.broadcasted_iota(jnp.int32, (1, tm), 1)
        xb = jnp.where(cols < m_total, x_ref[...], 0.0).astype(jnp.bfloat16)
        _phase0_work(xb)

    @pl.when(p == 1)
    def _phase1():
        @pl.when(j == 0)
        def _finalize_stats():
            count = jnp.float32(m_total)
            mean = sum_ref[...] / count
            var = jnp.maximum(sumsq_ref[...] / count - mean * mean, 0.0)
            inv = jax.lax.rsqrt(var + _BN_EPS)
            scale = gamma_ref[...] * inv
            shift = beta_ref[...] - mean * scale
            w2_ref[:, :w11g_ref.shape[1]] = (
                w11g_ref[...].astype(jnp.float32) * scale).astype(jnp.bfloat16)
            col = jax.lax.broadcasted_iota(jnp.int32, (shift.shape[0], _PAD), 1)
            w2_ref[:, w11g_ref.shape[1]:] = jnp.where(
                col == 0, shift, 0.0).astype(jnp.bfloat16)
        out_ref[...] = jnp.dot(w2_ref[...], xb_ref[j],
                               preferred_element_type=jnp.float32)


@jax.jit
def _forward(x30, x27, w10, b10, w11, gamma, beta):
    N, Cmid, H, W = x27.shape
    Cin = x30.shape[1]
    Cout = w11.shape[0]
    M = H * W
    tm = min(2048, pl.cdiv(M, 128) * 128)
    n_tiles = pl.cdiv(M, tm)

    x = x27.reshape(Cmid, M)
    x30v = x30.reshape(1, Cin)
    b10r = b10.reshape(1, Cmid)
    gammac = gamma.reshape(Cout, 1)
    betac = beta.reshape(Cout, 1)

    body = functools.partial(_fused_body, tm=tm, m_total=M, n_tiles=n_tiles)

    out = pl.pallas_call(
        body,
        out_shape=jax.ShapeDtypeStruct((Cout, M), jnp.float32),
        grid=(2, n_tiles),
        in_specs=[
            pl.BlockSpec((1, Cin), lambda p, j: (0, 0)),
            pl.BlockSpec((Cmid, Cin), lambda p, j: (0, 0)),
            pl.BlockSpec((1, Cmid), lambda p, j: (0, 0)),
            pl.BlockSpec((Cmid, tm),
                         lambda p, j: (0, (1 - p) * j + p * (n_tiles - 1))),
            pl.BlockSpec((Cout, Cmid), lambda p, j: (0, 0)),
            pl.BlockSpec((Cout, 1), lambda p, j: (0, 0)),
            pl.BlockSpec((Cout, 1), lambda p, j: (0, 0)),
        ],
        out_specs=pl.BlockSpec((Cout, tm), lambda p, j: (0, p * j)),
        scratch_shapes=[
            pltpu.VMEM((Cout, Cmid), jnp.bfloat16),
            pltpu.VMEM((n_tiles, Cmid + _PAD, tm), jnp.bfloat16),
            pltpu.VMEM((Cout, Cmid + _PAD), jnp.bfloat16),
            pltpu.VMEM((Cout, 1), jnp.float32),
            pltpu.VMEM((Cout, 1), jnp.float32),
        ],
        compiler_params=pltpu.CompilerParams(
            dimension_semantics=("arbitrary", "arbitrary"),
            vmem_limit_bytes=64 * 1024 * 1024),
    )(x30v, w10, b10r, x, w11, gammac, betac)

    return out.reshape(N, Cout, H, W)


def kernel(x30, x27, w10, b10, w11, gamma, beta):
    return _forward(x30, x27, w10, b10, w11, gamma, beta)


# speedup vs baseline: 1.0704x; 1.0704x over previous
import functools

import jax
import jax.numpy as jnp
from jax.experimental import pallas as pl
from jax.experimental.pallas import tpu as pltpu

_BN_EPS = 1e-5


def _fused_body(x30_ref, w10_ref, b10_ref, x_ref, w11_ref, gamma_ref, beta_ref,
                out_ref,
                w11g_ref, y_ref, sum_ref, sumsq_ref, scale_ref, shift_ref,
                *, tm, m_total, n_tiles):
    p = pl.program_id(0)
    j = pl.program_id(1)

    @pl.when(jnp.logical_and(p == 0, j == 0))
    def _init():
        g = jax.lax.dot_general(x30_ref[...], w10_ref[...],
                                (((1,), (1,)), ((), ())),
                                preferred_element_type=jnp.float32)
        gate = jax.nn.sigmoid(g + b10_ref[...])
        w11g_ref[...] = (w11_ref[...] * gate).astype(jnp.bfloat16)
        sum_ref[...] = jnp.zeros_like(sum_ref)
        sumsq_ref[...] = jnp.zeros_like(sumsq_ref)
        out_ref[...] = jnp.zeros_like(out_ref)

    def _accumulate(y):
        y_ref[j] = y.astype(jnp.bfloat16)
        sum_ref[...] += jnp.sum(y, axis=1, keepdims=True)
        sumsq_ref[...] += jnp.sum(y * y, axis=1, keepdims=True)

    @pl.when(jnp.logical_and(p == 0, j < n_tiles - 1))
    def _phase0_full():
        xb = x_ref[...].astype(jnp.bfloat16)
        _accumulate(jnp.dot(w11g_ref[...], xb, preferred_element_type=jnp.float32))

    @pl.when(jnp.logical_and(p == 0, j == n_tiles - 1))
    def _phase0_last():
        xb = x_ref[...].astype(jnp.bfloat16)
        y = jnp.dot(w11g_ref[...], xb, preferred_element_type=jnp.float32)
        cols = j * tm + jax.lax.broadcasted_iota(jnp.int32, (1, tm), 1)
        _accumulate(jnp.where(cols < m_total, y, 0.0))

    @pl.when(p == 1)
    def _phase1():
        @pl.when(j == 0)
        def _finalize_stats():
            count = jnp.float32(m_total)
            mean = sum_ref[...] / count
            var = jnp.maximum(sumsq_ref[...] / count - mean * mean, 0.0)
            inv = jax.lax.rsqrt(var + _BN_EPS)
            scale = gamma_ref[...] * inv
            scale_ref[...] = scale.astype(jnp.bfloat16)
            shift_ref[...] = (beta_ref[...] - mean * scale).astype(jnp.bfloat16)
        out_ref[...] = (y_ref[j] * scale_ref[...] +
                        shift_ref[...]).astype(jnp.float32)


@jax.jit
def _forward(x30, x27, w10, b10, w11, gamma, beta):
    N, Cmid, H, W = x27.shape
    Cin = x30.shape[1]
    Cout = w11.shape[0]
    M = H * W
    tm = min(2048, pl.cdiv(M, 128) * 128)
    n_tiles = pl.cdiv(M, tm)

    x = x27.reshape(Cmid, M)
    x30v = x30.reshape(1, Cin)
    b10r = b10.reshape(1, Cmid)
    gammac = gamma.reshape(Cout, 1)
    betac = beta.reshape(Cout, 1)

    body = functools.partial(_fused_body, tm=tm, m_total=M, n_tiles=n_tiles)

    out = pl.pallas_call(
        body,
        out_shape=jax.ShapeDtypeStruct((Cout, M), jnp.float32),
        grid=(2, n_tiles),
        in_specs=[
            pl.BlockSpec((1, Cin), lambda p, j: (0, 0)),
            pl.BlockSpec((Cmid, Cin), lambda p, j: (0, 0)),
            pl.BlockSpec((1, Cmid), lambda p, j: (0, 0)),
            pl.BlockSpec((Cmid, tm),
                         lambda p, j: (0, (1 - p) * j + p * (n_tiles - 1))),
            pl.BlockSpec((Cout, Cmid), lambda p, j: (0, 0)),
            pl.BlockSpec((Cout, 1), lambda p, j: (0, 0)),
            pl.BlockSpec((Cout, 1), lambda p, j: (0, 0)),
        ],
        out_specs=pl.BlockSpec((Cout, tm), lambda p, j: (0, p * j)),
        scratch_shapes=[
            pltpu.VMEM((Cout, Cmid), jnp.bfloat16),
            pltpu.VMEM((n_tiles, Cout, tm), jnp.bfloat16),
            pltpu.VMEM((Cout, 1), jnp.float32),
            pltpu.VMEM((Cout, 1), jnp.float32),
            pltpu.VMEM((Cout, 1), jnp.bfloat16),
            pltpu.VMEM((Cout, 1), jnp.bfloat16),
        ],
        compiler_params=pltpu.CompilerParams(
            dimension_semantics=("arbitrary", "arbitrary"),
            vmem_limit_bytes=64 * 1024 * 1024),
    )(x30v, w10, b10r, x, w11, gammac, betac)

    return out.reshape(N, Cout, H, W)


def kernel(x30, x27, w10, b10, w11, gamma, beta):
    return _forward(x30, x27, w10, b10, w11, gamma, beta)
